# baseline (speedup 1.0000x reference)
import functools

import jax
import jax.numpy as jnp
from jax import lax
from jax.experimental import pallas as pl
from jax.experimental.pallas import tpu as pltpu
from jax.experimental.pallas import tpu_sc as plsc

SEQ_LEN = 200
EMBED_DIM = 64
BATCH = 4096

NC, NS, L = 2, 16, 16
NW = NC * NS
SEQ_PER_W = BATCH // NW
G0, G1 = 128, SEQ_LEN - 128
SCALE = 8.0


@functools.partial(
    pl.kernel,
    out_type=jax.ShapeDtypeStruct((BATCH, SEQ_LEN, EMBED_DIM), jnp.float32),
    mesh=plsc.VectorSubcoreMesh(core_axis_name="c", subcore_axis_name="s"),
    compiler_params=pltpu.CompilerParams(use_tc_tiling_on_sc=False),
    scratch_types=[
        pltpu.VMEM((SEQ_LEN,), jnp.int32),
        pltpu.VMEM((SEQ_LEN, EMBED_DIM), jnp.float32),
        pltpu.VMEM((SEQ_LEN, EMBED_DIM), jnp.float32),
        pltpu.SemaphoreType.DMA,
    ],
)
def _embed_kernel(inputs_hbm, token_hbm, pos_hbm, out_hbm,
                  idx_v, rows_v, pos_v, sem):
    wid = lax.axis_index("s") * NC + lax.axis_index("c")
    base_seq = wid * SEQ_PER_W

    pltpu.sync_copy(pos_hbm, pos_v)

    def seq_body(s, carry):
        seq = base_seq + s
        pltpu.sync_copy(inputs_hbm.at[seq], idx_v)
        c0 = pltpu.async_copy(token_hbm.at[idx_v.at[pl.ds(0, G0)]],
                              rows_v.at[pl.ds(0, G0)], sem)
        c1 = pltpu.async_copy(token_hbm.at[idx_v.at[pl.ds(G0, G1)]],
                              rows_v.at[pl.ds(G0, G1)], sem)
        c0.wait()
        c1.wait()

        def row_body(r, carry2):
            for j in range(EMBED_DIM // L):
                sl = pl.ds(j * L, L)
                rows_v[r, sl] = rows_v[r, sl] * SCALE + pos_v[r, sl]
            return carry2

        lax.fori_loop(0, SEQ_LEN, row_body, 0, unroll=2)
        pltpu.sync_copy(rows_v, out_hbm.at[seq])
        return carry

    lax.fori_loop(0, SEQ_PER_W, seq_body, 0)


def kernel(inputs, token_table, pos_table):
    return _embed_kernel(inputs, token_table, pos_table)


# speedup vs baseline: 1.1298x; 1.1298x over previous
import functools

import jax
import jax.numpy as jnp
from jax import lax
from jax.experimental import pallas as pl
from jax.experimental.pallas import tpu as pltpu
from jax.experimental.pallas import tpu_sc as plsc

SEQ_LEN = 200
EMBED_DIM = 64
BATCH = 4096
TOTAL_ROWS = BATCH * SEQ_LEN

NC, NS, L = 2, 16, 16
NW = NC * NS
CHUNKS_PER_W = BATCH // NW
NBUF = 4
G0, G1 = 128, SEQ_LEN - 128
SCALE = 8.0


@functools.partial(
    pl.kernel,
    out_type=jax.ShapeDtypeStruct((TOTAL_ROWS, EMBED_DIM), jnp.float32),
    mesh=plsc.VectorSubcoreMesh(core_axis_name="c", subcore_axis_name="s"),
    compiler_params=pltpu.CompilerParams(use_tc_tiling_on_sc=False),
    scratch_types=[
        [pltpu.VMEM((SEQ_LEN,), jnp.int32) for _ in range(NBUF)],
        [pltpu.VMEM((SEQ_LEN, EMBED_DIM), jnp.float32) for _ in range(NBUF)],
        pltpu.VMEM((SEQ_LEN, EMBED_DIM), jnp.float32),
        [pltpu.SemaphoreType.DMA for _ in range(NBUF)],
        [pltpu.SemaphoreType.DMA for _ in range(NBUF)],
    ],
)
def _embed_kernel(inputs_hbm, token_hbm, pos_hbm, out_hbm,
                  idx_v, rows_v, pos_v, gsem, wsem):
    wid = lax.axis_index("s") * NC + lax.axis_index("c")
    base_row = wid * CHUNKS_PER_W * SEQ_LEN

    pltpu.sync_copy(pos_hbm, pos_v)

    def fire_gather(g, b):
        pltpu.sync_copy(inputs_hbm.at[pl.ds(base_row + g * SEQ_LEN, SEQ_LEN)],
                        idx_v[b])
        pltpu.async_copy(token_hbm.at[idx_v[b].at[pl.ds(0, G0)]],
                         rows_v[b].at[pl.ds(0, G0)], gsem[b])
        pltpu.async_copy(token_hbm.at[idx_v[b].at[pl.ds(G0, G1)]],
                         rows_v[b].at[pl.ds(G0, G1)], gsem[b])

    def wait_gather(b):
        pltpu.make_async_copy(token_hbm.at[idx_v[b].at[pl.ds(0, G0)]],
                              rows_v[b].at[pl.ds(0, G0)], gsem[b]).wait()
        pltpu.make_async_copy(token_hbm.at[idx_v[b].at[pl.ds(G0, G1)]],
                              rows_v[b].at[pl.ds(G0, G1)], gsem[b]).wait()

    def wb_descr(g, b):
        return pltpu.make_async_copy(
            rows_v[b], out_hbm.at[pl.ds(base_row + g * SEQ_LEN, SEQ_LEN)],
            wsem[b])

    fire_gather(0, 0)

    def outer(o, carry):
        for b in range(NBUF):
            g = o * NBUF + b
            nxt = g + 1
            nb = (b + 1) % NBUF

            @pl.when(jnp.logical_and(nxt < CHUNKS_PER_W, g >= NBUF - 1))
            def _():
                wb_descr(g - (NBUF - 1), nb).wait()

            @pl.when(nxt < CHUNKS_PER_W)
            def _():
                fire_gather(nxt, nb)

            wait_gather(b)

            def row_body(r, carry2):
                for j in range(EMBED_DIM // L):
                    sl = pl.ds(j * L, L)
                    rows_v[b][r, sl] = rows_v[b][r, sl] * SCALE + pos_v[r, sl]
                return carry2

            lax.fori_loop(0, SEQ_LEN, row_body, 0, unroll=2)
            pltpu.async_copy(rows_v[b],
                             out_hbm.at[pl.ds(base_row + g * SEQ_LEN, SEQ_LEN)],
                             wsem[b])
        return carry

    lax.fori_loop(0, CHUNKS_PER_W // NBUF, outer, 0)

    for b in range(NBUF):
        wb_descr(CHUNKS_PER_W - NBUF + b, b).wait()


def kernel(inputs, token_table, pos_table):
    out = _embed_kernel(inputs.reshape(TOTAL_ROWS), token_table, pos_table)
    return out.reshape(BATCH, SEQ_LEN, EMBED_DIM)
